# baseline (speedup 1.0000x reference)
import functools

import jax
import jax.numpy as jnp
from jax import lax
from jax.experimental import pallas as pl
from jax.experimental.pallas import tpu as pltpu
from jax.experimental.pallas import tpu_sc as plsc

VOCAB = 1000
DIM = 64
BATCH = 4096
SEQ_LEN = 200
TOTAL = BATCH * SEQ_LEN

NUM_CORES = 2
NUM_SUBCORES = 16
NUM_WORKERS = NUM_CORES * NUM_SUBCORES

LANES = 16
CHUNK = 256
GROUPS = CHUNK // LANES
PER_WORKER = TOTAL // NUM_WORKERS
STEPS = PER_WORKER // CHUNK


def _emb_body(table_hbm, x_hbm, out_hbm, table_v, idx_v, rows_v, sem_o):
    c = lax.axis_index("c")
    s = lax.axis_index("s")
    wid = s * NUM_CORES + c
    base = wid * PER_WORKER

    pltpu.sync_copy(table_hbm, table_v)
    pltpu.sync_copy(x_hbm.at[pl.ds(base, PER_WORKER)], idx_v)

    lane_off = lax.iota(jnp.int32, LANES) * DIM

    def gather_chunk(ichunk, b):
        def rowgroup(g, carry):
            idx16 = idx_v[pl.ds(ichunk * CHUNK + g * LANES, LANES)]
            src_base = idx16 * DIM
            dst_base = lane_off + g * (LANES * DIM)
            for d in range(DIM):
                val = plsc.load_gather(table_v, [src_base + d])
                plsc.store_scatter(rows_v.at[b], [dst_base + d], val)
            return carry

        lax.fori_loop(0, GROUPS, rowgroup, 0)

    def fire_out(ichunk, b):
        pltpu.async_copy(
            rows_v.at[b],
            out_hbm.at[pl.ds((base + ichunk * CHUNK) * DIM, CHUNK * DIM)],
            sem_o.at[b],
        )

    def wait_out(b):
        pltpu.make_async_copy(
            rows_v.at[b], out_hbm.at[pl.ds(0, CHUNK * DIM)], sem_o.at[b]
        ).wait()

    gather_chunk(0, 0)
    fire_out(0, 0)
    gather_chunk(1, 1)
    fire_out(1, 1)

    def outer(g, carry):
        for off in range(2):
            i = 2 * g + 2 + off
            b = off
            wait_out(b)
            gather_chunk(i, b)
            fire_out(i, b)
        return carry

    lax.fori_loop(0, (STEPS - 2) // 2, outer, 0)

    wait_out(0)
    wait_out(1)


@functools.partial(
    pl.kernel,
    mesh=plsc.VectorSubcoreMesh(core_axis_name="c", subcore_axis_name="s"),
    out_type=jax.ShapeDtypeStruct((TOTAL * DIM,), jnp.float32),
    scratch_types=[
        pltpu.VMEM((VOCAB * DIM,), jnp.float32),
        pltpu.VMEM((PER_WORKER,), jnp.int32),
        pltpu.VMEM((2, CHUNK * DIM), jnp.float32),
        pltpu.SemaphoreType.DMA((2,)),
    ],
    compiler_params=pltpu.CompilerParams(
        use_tc_tiling_on_sc=False, needs_layout_passes=False
    ),
)
def _emb_call(table_hbm, x_hbm, out_hbm, table_v, idx_v, rows_v, sem_o):
    _emb_body(table_hbm, x_hbm, out_hbm, table_v, idx_v, rows_v, sem_o)


def kernel(x, table):
    xf = x.reshape(TOTAL).astype(jnp.int32)
    out = _emb_call(table.reshape(VOCAB * DIM), xf)
    return out.reshape(BATCH, SEQ_LEN, DIM)


# speedup vs baseline: 3.9360x; 3.9360x over previous
import functools

import jax
import jax.numpy as jnp
from jax import lax
from jax.experimental import pallas as pl
from jax.experimental.pallas import tpu as pltpu
from jax.experimental.pallas import tpu_sc as plsc

VOCAB = 1000
DIM = 64
BATCH = 4096
SEQ_LEN = 200
TOTAL = BATCH * SEQ_LEN

NUM_CORES = 2
NUM_SUBCORES = 16
NUM_WORKERS = NUM_CORES * NUM_SUBCORES

LANES = 16
CHUNK = 256
GROUPS = CHUNK // LANES
PER_WORKER = TOTAL // NUM_WORKERS
STEPS = PER_WORKER // CHUNK


def _emb_body(table_hbm, x_hbm, out_hbm, table_v, idx_v, rows_v, sem_o):
    c = lax.axis_index("c")
    s = lax.axis_index("s")
    wid = s * NUM_CORES + c
    base = wid * PER_WORKER

    pltpu.sync_copy(table_hbm, table_v)
    pltpu.sync_copy(x_hbm.at[pl.ds(base, PER_WORKER)], idx_v)

    def gather_chunk(ichunk, b):
        base_i = ichunk * CHUNK

        @plsc.parallel_loop(0, CHUNK, step=LANES)
        def _rowgroup(r0):
            idx16 = idx_v[pl.ds(base_i + r0, LANES)] * DIM
            for r in range(LANES):
                src = idx16[r]
                for q in range(DIM // LANES):
                    rows_v[b, pl.ds((r0 + r) * DIM + q * LANES, LANES)] = table_v[
                        pl.ds(src + q * LANES, LANES)
                    ]

    def fire_out(ichunk, b):
        pltpu.async_copy(
            rows_v.at[b],
            out_hbm.at[pl.ds((base + ichunk * CHUNK) * DIM, CHUNK * DIM)],
            sem_o.at[b],
        )

    def wait_out(b):
        pltpu.make_async_copy(
            rows_v.at[b], out_hbm.at[pl.ds(0, CHUNK * DIM)], sem_o.at[b]
        ).wait()

    gather_chunk(0, 0)
    fire_out(0, 0)
    gather_chunk(1, 1)
    fire_out(1, 1)

    def outer(g, carry):
        for off in range(2):
            i = 2 * g + 2 + off
            b = off
            wait_out(b)
            gather_chunk(i, b)
            fire_out(i, b)
        return carry

    lax.fori_loop(0, (STEPS - 2) // 2, outer, 0)

    wait_out(0)
    wait_out(1)


@functools.partial(
    pl.kernel,
    mesh=plsc.VectorSubcoreMesh(core_axis_name="c", subcore_axis_name="s"),
    out_type=jax.ShapeDtypeStruct((TOTAL * DIM,), jnp.float32),
    scratch_types=[
        pltpu.VMEM((VOCAB * DIM,), jnp.float32),
        pltpu.VMEM((PER_WORKER,), jnp.int32),
        pltpu.VMEM((2, CHUNK * DIM), jnp.float32),
        pltpu.SemaphoreType.DMA((2,)),
    ],
    compiler_params=pltpu.CompilerParams(
        use_tc_tiling_on_sc=False, needs_layout_passes=False
    ),
)
def _emb_call(table_hbm, x_hbm, out_hbm, table_v, idx_v, rows_v, sem_o):
    _emb_body(table_hbm, x_hbm, out_hbm, table_v, idx_v, rows_v, sem_o)


def kernel(x, table):
    xf = x.reshape(TOTAL).astype(jnp.int32)
    out = _emb_call(table.reshape(VOCAB * DIM), xf)
    return out.reshape(BATCH, SEQ_LEN, DIM)


# speedup vs baseline: 3.9479x; 1.0030x over previous
import functools

import jax
import jax.numpy as jnp
from jax import lax
from jax.experimental import pallas as pl
from jax.experimental.pallas import tpu as pltpu
from jax.experimental.pallas import tpu_sc as plsc

VOCAB = 1000
DIM = 64
BATCH = 4096
SEQ_LEN = 200
TOTAL = BATCH * SEQ_LEN

NUM_CORES = 2
NUM_SUBCORES = 16
NUM_WORKERS = NUM_CORES * NUM_SUBCORES

LANES = 16
CHUNK = 256
GROUPS = CHUNK // LANES
PER_WORKER = TOTAL // NUM_WORKERS
STEPS = PER_WORKER // CHUNK


def _emb_body(table_hbm, x_hbm, out_hbm, table_v, idx_v, rows_v, sem_o, sem_st):
    c = lax.axis_index("c")
    s = lax.axis_index("s")
    wid = s * NUM_CORES + c
    base = wid * PER_WORKER

    pltpu.async_copy(table_hbm, table_v, sem_st.at[0])
    pltpu.async_copy(x_hbm.at[pl.ds(base, PER_WORKER)], idx_v, sem_st.at[1])
    pltpu.make_async_copy(table_hbm, table_v, sem_st.at[0]).wait()
    pltpu.make_async_copy(
        x_hbm.at[pl.ds(0, PER_WORKER)], idx_v, sem_st.at[1]
    ).wait()

    def gather_chunk(ichunk, b):
        base_i = ichunk * CHUNK

        @plsc.parallel_loop(0, CHUNK, step=LANES)
        def _rowgroup(r0):
            idx16 = idx_v[pl.ds(base_i + r0, LANES)] * DIM
            for r in range(LANES):
                src = idx16[r]
                for q in range(DIM // LANES):
                    rows_v[b, pl.ds((r0 + r) * DIM + q * LANES, LANES)] = table_v[
                        pl.ds(src + q * LANES, LANES)
                    ]

    def fire_out(ichunk, b):
        pltpu.async_copy(
            rows_v.at[b],
            out_hbm.at[pl.ds((base + ichunk * CHUNK) * DIM, CHUNK * DIM)],
            sem_o.at[b],
        )

    def wait_out(b):
        pltpu.make_async_copy(
            rows_v.at[b], out_hbm.at[pl.ds(0, CHUNK * DIM)], sem_o.at[b]
        ).wait()

    gather_chunk(0, 0)
    fire_out(0, 0)
    gather_chunk(1, 1)
    fire_out(1, 1)

    def outer(g, carry):
        for off in range(2):
            i = 2 * g + 2 + off
            b = off
            wait_out(b)
            gather_chunk(i, b)
            fire_out(i, b)
        return carry

    lax.fori_loop(0, (STEPS - 2) // 2, outer, 0)

    wait_out(0)
    wait_out(1)


@functools.partial(
    pl.kernel,
    mesh=plsc.VectorSubcoreMesh(core_axis_name="c", subcore_axis_name="s"),
    out_type=jax.ShapeDtypeStruct((TOTAL * DIM,), jnp.float32),
    scratch_types=[
        pltpu.VMEM((VOCAB * DIM,), jnp.float32),
        pltpu.VMEM((PER_WORKER,), jnp.int32),
        pltpu.VMEM((2, CHUNK * DIM), jnp.float32),
        pltpu.SemaphoreType.DMA((2,)),
        pltpu.SemaphoreType.DMA((2,)),
    ],
    compiler_params=pltpu.CompilerParams(
        use_tc_tiling_on_sc=False, needs_layout_passes=False
    ),
)
def _emb_call(table_hbm, x_hbm, out_hbm, table_v, idx_v, rows_v, sem_o, sem_st):
    _emb_body(table_hbm, x_hbm, out_hbm, table_v, idx_v, rows_v, sem_o, sem_st)


def kernel(x, table):
    xf = x.reshape(TOTAL).astype(jnp.int32)
    out = _emb_call(table.reshape(VOCAB * DIM), xf)
    return out.reshape(BATCH, SEQ_LEN, DIM)
